# Initial kernel scaffold; baseline (speedup 1.0000x reference)
import jax
import jax.numpy as jnp
from jax.experimental import pallas as pl


def kernel(n_feat, e_feat, edge_index, W0, W1, W2, W3, W4, gamma_e, beta_e, gamma_n, beta_n):
    raise NotImplementedError("write your pallas kernel here")


# speedup vs baseline: 1.9372x; 1.9372x over previous
import functools

import jax
import jax.numpy as jnp
from jax import lax
from jax.experimental import pallas as pl
from jax.experimental.pallas import tpu as pltpu
from jax.experimental.pallas import tpu_sc as plsc

N = 10000
E = 160000
D = 256
H = 128
EPS = 1e-5

C = 80
TILES = 16
EDGES_PER_TILE = E // (TILES)
CHUNKS = EDGES_PER_TILE // C
ROWS_PER_TILE = N // TILES


def _mm_kernel(x_ref, w_ref, o_ref):
    o_ref[...] = jnp.dot(x_ref[...], w_ref[...],
                         preferred_element_type=jnp.float32)


def _node_matmuls(n_feat, wnt):
    return pl.pallas_call(
        _mm_kernel,
        grid=(25,),
        in_specs=[pl.BlockSpec((400, D), lambda i: (i, 0)),
                  pl.BlockSpec((D, 4 * D), lambda i: (0, 0))],
        out_specs=pl.BlockSpec((400, 4 * D), lambda i: (i, 0)),
        out_shape=jax.ShapeDtypeStruct((N, 4 * D), jnp.float32),
    )(n_feat, wnt)


def _edge_mm_kernel(x_ref, w_ref, w4e_ref, ex_ref):
    x = x_ref[...]
    w4e_ref[...] = jnp.dot(x, w_ref[...], preferred_element_type=jnp.float32)
    ex_ref[...] = jnp.exp(x)


def _edge_matmul_exp(e_feat, w4t):
    return pl.pallas_call(
        _edge_mm_kernel,
        grid=(160,),
        in_specs=[pl.BlockSpec((1000, D), lambda i: (i, 0)),
                  pl.BlockSpec((D, D), lambda i: (0, 0))],
        out_specs=[pl.BlockSpec((1000, D), lambda i: (i, 0)),
                   pl.BlockSpec((1000, D), lambda i: (i, 0))],
        out_shape=[jax.ShapeDtypeStruct((E, D), jnp.float32),
                   jax.ShapeDtypeStruct((E, D), jnp.float32)],
    )(e_feat, w4t)


def _estats_kernel(ep_ref, w4_ref, st_ref):
    s = ep_ref[...] + w4_ref[...]
    ps = jnp.sum(s, axis=0)
    pq = jnp.sum(s * s, axis=0)
    z = jnp.zeros((6, D), jnp.float32)
    st_ref[...] = jnp.concatenate([ps[None], pq[None], z], axis=0)


def _edge_stats(e_pre, w4e):
    return pl.pallas_call(
        _estats_kernel,
        grid=(160,),
        in_specs=[pl.BlockSpec((1000, D), lambda i: (i, 0)),
                  pl.BlockSpec((1000, D), lambda i: (i, 0))],
        out_specs=pl.BlockSpec((8, D), lambda i: (i, 0)),
        out_shape=jax.ShapeDtypeStruct((160 * 8, D), jnp.float32),
    )(e_pre, w4e)


def _newe_kernel(st_ref, ep_ref, w4_ref, ef_ref, g_ref, b_ref, o_ref):
    st = jnp.sum(st_ref[...].reshape(160, 8, D), axis=0)
    mean = st[0:1] / E
    var = st[1:2] / E - mean * mean
    inv = lax.rsqrt(var + EPS)
    s = ep_ref[...] + w4_ref[...]
    xn = (s - mean) * inv * g_ref[...] + b_ref[...]
    o_ref[...] = jnp.maximum(xn, 0.0) + ef_ref[...]


def _edge_update(stats, e_pre, w4e, e_feat, gamma_e, beta_e):
    return pl.pallas_call(
        _newe_kernel,
        grid=(160,),
        in_specs=[pl.BlockSpec((160 * 8, D), lambda i: (0, 0)),
                  pl.BlockSpec((1000, D), lambda i: (i, 0)),
                  pl.BlockSpec((1000, D), lambda i: (i, 0)),
                  pl.BlockSpec((1000, D), lambda i: (i, 0)),
                  pl.BlockSpec((1, D), lambda i: (0, 0)),
                  pl.BlockSpec((1, D), lambda i: (0, 0))],
        out_specs=pl.BlockSpec((1000, D), lambda i: (i, 0)),
        out_shape=jax.ShapeDtypeStruct((E, D), jnp.float32),
    )(stats, e_pre, w4e, e_feat, gamma_e, beta_e)


def _recip_kernel(x_ref, o_ref):
    o_ref[...] = 1.0 / x_ref[...]


def _recip(x):
    return pl.pallas_call(
        _recip_kernel,
        grid=(1,),
        in_specs=[pl.BlockSpec(x.shape, lambda i: (0, 0))],
        out_specs=pl.BlockSpec(x.shape, lambda i: (0, 0)),
        out_shape=jax.ShapeDtypeStruct(x.shape, jnp.float32),
    )(x)


def _newh_kernel(nt_ref, w2_ref, nf_ref, g_ref, b_ref, o_ref):
    s = nt_ref[...] + w2_ref[...]
    mean = jnp.mean(s, axis=0, keepdims=True)
    var = jnp.mean(s * s, axis=0, keepdims=True) - mean * mean
    inv = lax.rsqrt(var + EPS)
    xn = (s - mean) * inv * g_ref[...] + b_ref[...]
    o_ref[...] = jnp.maximum(xn, 0.0) + nf_ref[...]


def _node_update(n_tmp, w2h, n_feat, gamma_n, beta_n):
    return pl.pallas_call(
        _newh_kernel,
        grid=(1,),
        in_specs=[pl.BlockSpec((N, D), lambda i: (0, 0)),
                  pl.BlockSpec((N, D), lambda i: (0, 0)),
                  pl.BlockSpec((N, D), lambda i: (0, 0)),
                  pl.BlockSpec((1, D), lambda i: (0, 0)),
                  pl.BlockSpec((1, D), lambda i: (0, 0))],
        out_specs=pl.BlockSpec((N, D), lambda i: (0, 0)),
        out_shape=jax.ShapeDtypeStruct((N, D), jnp.float32),
    )(n_tmp, w2h, n_feat, gamma_n, beta_n)


_MESH = plsc.VectorSubcoreMesh(core_axis_name="c", subcore_axis_name="s")


def _adjust_idx(dst_ref, src_ref, off):
    for v in range(C // 16):
        sl = pl.ds(v * 16, 16)
        dst_ref[sl] = src_ref[sl] + off


def _sc_edge_kernel(src_hbm, dst_hbm, w0_hbm, w1_hbm, ex_hbm, zero_hbm,
                    epre_hbm, denom_hbm,
                    acc, srcv, dstv, srcv2, buf0, buf1):
    c = lax.axis_index("c")
    s = lax.axis_index("s")
    half = c * H

    @pl.when(s == 0)
    def _():
        pltpu.sync_copy(zero_hbm, acc)
    plsc.subcore_barrier()

    def chunk(k, _):
        e0 = s * EDGES_PER_TILE + k * C
        pltpu.sync_copy(src_hbm.at[pl.ds(e0, C)], srcv)
        pltpu.sync_copy(dst_hbm.at[pl.ds(e0, C)], dstv)
        _adjust_idx(srcv2, srcv, c * N)
        pltpu.sync_copy(w0_hbm.at[srcv2], buf0)
        _adjust_idx(srcv2, dstv, c * N)
        pltpu.sync_copy(w1_hbm.at[srcv2], buf1)

        def row_add(r, _):
            for j in range(H // 16):
                sl = pl.ds(j * 16, 16)
                buf0[r, sl] = buf0[r, sl] + buf1[r, sl]
            return 0
        lax.fori_loop(0, C, row_add, 0)
        pltpu.sync_copy(buf0, epre_hbm.at[pl.ds(e0, C), pl.ds(half, H)])

        pltpu.sync_copy(ex_hbm.at[pl.ds(e0, C), pl.ds(half, H)], buf1)
        pltpu.sync_copy(buf1, acc.at[dstv], add=True)
        return 0

    lax.fori_loop(0, CHUNKS, chunk, 0)
    plsc.subcore_barrier()

    @pl.when(s == 0)
    def _():
        pltpu.sync_copy(acc, denom_hbm.at[c])


def _sc_edge(src, dst, w0cat, w1cat, ex, zeros):
    return pl.kernel(
        _sc_edge_kernel,
        out_type=[jax.ShapeDtypeStruct((E, D), jnp.float32),
                  jax.ShapeDtypeStruct((2, N, H), jnp.float32)],
        mesh=_MESH,
        scratch_types=[
            pltpu.VMEM_SHARED((N, H), jnp.float32),
            pltpu.VMEM((C,), jnp.int32),
            pltpu.VMEM((C,), jnp.int32),
            pltpu.VMEM((C,), jnp.int32),
            pltpu.VMEM((C, H), jnp.float32),
            pltpu.VMEM((C, H), jnp.float32),
        ],
    )(src, dst, w0cat, w1cat, ex, zeros)


def _sc_node_kernel(src_hbm, dst_hbm, w3_hbm, rd_hbm, ex_hbm, zero_hbm,
                    ntmp_hbm,
                    acc, srcv, dstv, idx2, bufw, bufr, bufe):
    c = lax.axis_index("c")
    s = lax.axis_index("s")
    half = c * H

    @pl.when(s == 0)
    def _():
        pltpu.sync_copy(zero_hbm, acc)
    plsc.subcore_barrier()

    def chunk(k, _):
        e0 = s * EDGES_PER_TILE + k * C
        pltpu.sync_copy(src_hbm.at[pl.ds(e0, C)], srcv)
        pltpu.sync_copy(dst_hbm.at[pl.ds(e0, C)], dstv)
        _adjust_idx(idx2, srcv, c * N)
        pltpu.sync_copy(w3_hbm.at[idx2], bufw)
        _adjust_idx(idx2, dstv, c * N)
        pltpu.sync_copy(rd_hbm.at[idx2], bufr)
        pltpu.sync_copy(ex_hbm.at[pl.ds(e0, C), pl.ds(half, H)], bufe)

        def row_mul(r, _):
            for j in range(H // 16):
                sl = pl.ds(j * 16, 16)
                bufw[r, sl] = bufw[r, sl] * bufe[r, sl] * bufr[r, sl]
            return 0
        lax.fori_loop(0, C, row_mul, 0)
        pltpu.sync_copy(bufw, acc.at[dstv], add=True)
        return 0

    lax.fori_loop(0, CHUNKS, chunk, 0)
    plsc.subcore_barrier()

    @pl.when(s == 0)
    def _():
        pltpu.sync_copy(acc, ntmp_hbm.at[c])


def _sc_node(src, dst, w3cat, rdcat, ex, zeros):
    return pl.kernel(
        _sc_node_kernel,
        out_type=jax.ShapeDtypeStruct((2, N, H), jnp.float32),
        mesh=_MESH,
        scratch_types=[
            pltpu.VMEM_SHARED((N, H), jnp.float32),
            pltpu.VMEM((C,), jnp.int32),
            pltpu.VMEM((C,), jnp.int32),
            pltpu.VMEM((C,), jnp.int32),
            pltpu.VMEM((C, H), jnp.float32),
            pltpu.VMEM((C, H), jnp.float32),
            pltpu.VMEM((C, H), jnp.float32),
        ],
    )(src, dst, w3cat, rdcat, ex, zeros)


def _halves_cat(x):
    return jnp.concatenate([x[:, :H], x[:, H:]], axis=0)


def kernel(n_feat, e_feat, edge_index, W0, W1, W2, W3, W4,
           gamma_e, beta_e, gamma_n, beta_n):
    src = edge_index[0]
    dst = edge_index[1]

    wnt = jnp.concatenate([W0, W1, W2, W3], axis=0).T
    hcat = _node_matmuls(n_feat, wnt)
    w0h, w1h, w2h, w3h = (hcat[:, :D], hcat[:, D:2 * D],
                          hcat[:, 2 * D:3 * D], hcat[:, 3 * D:])

    w4e, ex = _edge_matmul_exp(e_feat, W4.T)

    zeros = jnp.zeros((N, H), jnp.float32)
    e_pre, denom = _sc_edge(src, dst, _halves_cat(w0h), _halves_cat(w1h),
                            ex, zeros)

    stats = _edge_stats(e_pre, w4e)
    new_e = _edge_update(stats, e_pre, w4e, e_feat,
                         gamma_e.reshape(1, D), beta_e.reshape(1, D))

    rdcat = _recip(denom.reshape(2 * N, H))
    ntmp_h = _sc_node(src, dst, _halves_cat(w3h), rdcat, ex, zeros)
    n_tmp = ntmp_h.transpose(1, 0, 2).reshape(N, D)

    new_h = _node_update(n_tmp, w2h, n_feat,
                         gamma_n.reshape(1, D), beta_n.reshape(1, D))
    return (new_h, new_e)


# speedup vs baseline: 2.8387x; 1.4654x over previous
import functools

import jax
import jax.numpy as jnp
from jax import lax
from jax.experimental import pallas as pl
from jax.experimental.pallas import tpu as pltpu
from jax.experimental.pallas import tpu_sc as plsc

N = 10000
E = 160000
D = 256
H = 128
EPS = 1e-5

C = 64
TILES = 16
EDGES_PER_TILE = 10112
EP = EDGES_PER_TILE * TILES
P = EP - E
CHUNKS = EDGES_PER_TILE // C
NACC = N + 16


def _mm_kernel(x_ref, w_ref, o_ref):
    o_ref[...] = jnp.dot(x_ref[...], w_ref[...],
                         preferred_element_type=jnp.float32)


def _node_matmuls(n_feat, wnt):
    return pl.pallas_call(
        _mm_kernel,
        grid=(25,),
        in_specs=[pl.BlockSpec((400, D), lambda i: (i, 0)),
                  pl.BlockSpec((D, 4 * D), lambda i: (0, 0))],
        out_specs=pl.BlockSpec((400, 4 * D), lambda i: (i, 0)),
        out_shape=jax.ShapeDtypeStruct((N, 4 * D), jnp.float32),
    )(n_feat, wnt)


def _edge_mm_kernel(x_ref, w_ref, w4e_ref, ex_ref):
    x = x_ref[...]
    w4e_ref[...] = jnp.dot(x, w_ref[...], preferred_element_type=jnp.float32)
    ex_ref[...] = jnp.exp(x)


def _edge_matmul_exp(e_feat, w4t):
    return pl.pallas_call(
        _edge_mm_kernel,
        grid=(160,),
        in_specs=[pl.BlockSpec((1000, D), lambda i: (i, 0)),
                  pl.BlockSpec((D, D), lambda i: (0, 0))],
        out_specs=[pl.BlockSpec((1000, D), lambda i: (i, 0)),
                   pl.BlockSpec((1000, D), lambda i: (i, 0))],
        out_shape=[jax.ShapeDtypeStruct((E, D), jnp.float32),
                   jax.ShapeDtypeStruct((EP, D), jnp.float32)],
    )(e_feat, w4t)


def _estats_kernel(ep_ref, w4_ref, st_ref):
    s = ep_ref[...] + w4_ref[...]
    ps = jnp.sum(s, axis=0)
    pq = jnp.sum(s * s, axis=0)
    z = jnp.zeros((6, D), jnp.float32)
    st_ref[...] = jnp.concatenate([ps[None], pq[None], z], axis=0)


def _edge_stats(e_pre, w4e):
    return pl.pallas_call(
        _estats_kernel,
        grid=(160,),
        in_specs=[pl.BlockSpec((1000, D), lambda i: (i, 0)),
                  pl.BlockSpec((1000, D), lambda i: (i, 0))],
        out_specs=pl.BlockSpec((8, D), lambda i: (i, 0)),
        out_shape=jax.ShapeDtypeStruct((160 * 8, D), jnp.float32),
    )(e_pre, w4e)


def _newe_kernel(st_ref, ep_ref, w4_ref, ef_ref, g_ref, b_ref, o_ref):
    st = jnp.sum(st_ref[...].reshape(160, 8, D), axis=0)
    mean = st[0:1] / E
    var = st[1:2] / E - mean * mean
    inv = lax.rsqrt(var + EPS)
    s = ep_ref[...] + w4_ref[...]
    xn = (s - mean) * inv * g_ref[...] + b_ref[...]
    o_ref[...] = jnp.maximum(xn, 0.0) + ef_ref[...]


def _edge_update(stats, e_pre, w4e, e_feat, gamma_e, beta_e):
    return pl.pallas_call(
        _newe_kernel,
        grid=(160,),
        in_specs=[pl.BlockSpec((160 * 8, D), lambda i: (0, 0)),
                  pl.BlockSpec((1000, D), lambda i: (i, 0)),
                  pl.BlockSpec((1000, D), lambda i: (i, 0)),
                  pl.BlockSpec((1000, D), lambda i: (i, 0)),
                  pl.BlockSpec((1, D), lambda i: (0, 0)),
                  pl.BlockSpec((1, D), lambda i: (0, 0))],
        out_specs=pl.BlockSpec((1000, D), lambda i: (i, 0)),
        out_shape=jax.ShapeDtypeStruct((E, D), jnp.float32),
    )(stats, e_pre, w4e, e_feat, gamma_e, beta_e)


def _recip_kernel(x_ref, o_ref):
    o_ref[...] = 1.0 / x_ref[...]


def _recip(x):
    return pl.pallas_call(
        _recip_kernel,
        grid=(1,),
        in_specs=[pl.BlockSpec(x.shape, lambda i: (0, 0))],
        out_specs=pl.BlockSpec(x.shape, lambda i: (0, 0)),
        out_shape=jax.ShapeDtypeStruct(x.shape, jnp.float32),
    )(x)


def _newh_kernel(nt_ref, w2_ref, nf_ref, g_ref, b_ref, o_ref):
    s = nt_ref[...] + w2_ref[...]
    mean = jnp.mean(s, axis=0, keepdims=True)
    var = jnp.mean(s * s, axis=0, keepdims=True) - mean * mean
    inv = lax.rsqrt(var + EPS)
    xn = (s - mean) * inv * g_ref[...] + b_ref[...]
    o_ref[...] = jnp.maximum(xn, 0.0) + nf_ref[...]


def _node_update(n_tmp, w2h, n_feat, gamma_n, beta_n):
    return pl.pallas_call(
        _newh_kernel,
        grid=(1,),
        in_specs=[pl.BlockSpec((N, D), lambda i: (0, 0)),
                  pl.BlockSpec((N, D), lambda i: (0, 0)),
                  pl.BlockSpec((N, D), lambda i: (0, 0)),
                  pl.BlockSpec((1, D), lambda i: (0, 0)),
                  pl.BlockSpec((1, D), lambda i: (0, 0))],
        out_specs=pl.BlockSpec((N, D), lambda i: (0, 0)),
        out_shape=jax.ShapeDtypeStruct((N, D), jnp.float32),
    )(n_tmp, w2h, n_feat, gamma_n, beta_n)


_MESH = plsc.VectorSubcoreMesh(core_axis_name="c", subcore_axis_name="s")

PAIRS = CHUNKS // 2


def _copy_idx(dst_ref, src_ref):
    for v in range(C // 16):
        sl = pl.ds(v * 16, 16)
        dst_ref[sl] = src_ref[sl]


class _EdgePipe:
    def __init__(self, c, s, sadj, dadj, dstr, ta_hbm, tb_hbm, ex_hbm,
                 sa, da, dv, dv2, b0, b1, be, isem, dsem, osem, acc):
        self.c, self.s = c, s
        self.sadj, self.dadj, self.dstr = sadj, dadj, dstr
        self.ta, self.tb, self.ex = ta_hbm, tb_hbm, ex_hbm
        self.sa, self.da, self.dv, self.dv2 = sa, da, dv, dv2
        self.b0, self.b1, self.be = b0, b1, be
        self.isem, self.dsem, self.osem = isem, dsem, osem
        self.acc = acc

    def _e0(self, k):
        return self.s * EDGES_PER_TILE + k * C

    def _i_descs(self, k, b):
        e0 = self._e0(k)
        ge = self.c * EP + e0
        return [
            (self.sadj.at[pl.ds(ge, C)], self.sa.at[b], self.isem.at[b, 0]),
            (self.dadj.at[pl.ds(ge, C)], self.da.at[b], self.isem.at[b, 1]),
            (self.dstr.at[pl.ds(e0, C)], self.dv.at[b], self.isem.at[b, 2]),
        ]

    def _d_descs(self, k, b):
        e0 = self._e0(k)
        half = self.c * H
        return [
            (self.ta.at[self.sa.at[b]], self.b0.at[b], self.dsem.at[b, 0]),
            (self.tb.at[self.da.at[b]], self.b1.at[b], self.dsem.at[b, 1]),
            (self.ex.at[pl.ds(e0, C), pl.ds(half, H)], self.be.at[b],
             self.dsem.at[b, 2]),
        ]

    def i_start(self, k, b):
        for sd in self._i_descs(k, b):
            pltpu.async_copy(*sd)

    def i_wait(self, k, b):
        for sd in self._i_descs(k, b):
            pltpu.make_async_copy(*sd).wait()

    def d_start(self, k, b):
        for sd in self._d_descs(k, b):
            pltpu.async_copy(*sd)

    def d_wait(self, k, b):
        for sd in self._d_descs(k, b):
            pltpu.make_async_copy(*sd).wait()

    def run(self):
        self.i_start(0, 0)

        def pair(g, _):
            k0 = 2 * g
            self.i_wait(k0, 0)

            @pl.when(g >= 1)
            def _():
                self.o_wait(k0 - 2, 0)
            self.d_start(k0, 0)

            @pl.when(g >= 1)
            def _():
                self.d_wait(k0 - 1, 1)
                self.comp(1)
                self.o_go(k0 - 1, 1)
            self.i_start(k0 + 1, 1)

            k1 = k0 + 1
            self.i_wait(k1, 1)

            @pl.when(g >= 1)
            def _():
                self.o_wait(k1 - 2, 1)
            self.d_start(k1, 1)

            self.d_wait(k0, 0)
            self.comp(0)
            self.o_go(k0, 0)

            @pl.when(k1 + 1 < CHUNKS)
            def _():
                self.i_start(k1 + 1, 0)
            return 0

        lax.fori_loop(0, PAIRS, pair, 0)
        self.d_wait(CHUNKS - 1, 1)
        self.comp(1)
        self.o_go(CHUNKS - 1, 1)
        self.o_wait(CHUNKS - 2, 0)
        self.o_wait(CHUNKS - 1, 1)


class _S1Pipe(_EdgePipe):
    def comp(self, b):
        _copy_idx(self.dv2.at[b], self.dv.at[b])

        @plsc.parallel_loop(0, C, unroll=2)
        def _(r):
            for j in range(H // 16):
                sl = pl.ds(j * 16, 16)
                self.b0[b, r, sl] = self.b0[b, r, sl] + self.b1[b, r, sl]

    def o_go(self, k, b):
        e0 = self._e0(k)
        half = self.c * H
        pltpu.async_copy(self.b0.at[b],
                         self.epre.at[pl.ds(e0, C), pl.ds(half, H)],
                         self.osem.at[b, 0])
        pltpu.async_copy(self.be.at[b], self.acc.at[self.dv2.at[b]],
                         self.osem.at[b, 1], add=True)

    def o_wait(self, k, b):
        e0 = self._e0(k)
        half = self.c * H
        pltpu.make_async_copy(self.b0.at[b],
                              self.epre.at[pl.ds(e0, C), pl.ds(half, H)],
                              self.osem.at[b, 0]).wait()
        pltpu.make_async_copy(self.be.at[b], self.acc.at[self.dv2.at[b]],
                              self.osem.at[b, 1]).wait()


class _S2Pipe(_EdgePipe):
    def comp(self, b):
        _copy_idx(self.dv2.at[b], self.dv.at[b])

        @plsc.parallel_loop(0, C, unroll=2)
        def _(r):
            for j in range(H // 16):
                sl = pl.ds(j * 16, 16)
                self.b0[b, r, sl] = (self.b0[b, r, sl] * self.be[b, r, sl]
                                     * self.b1[b, r, sl])

    def o_go(self, k, b):
        pltpu.async_copy(self.b0.at[b], self.acc.at[self.dv2.at[b]],
                         self.osem.at[b, 0], add=True)

    def o_wait(self, k, b):
        pltpu.make_async_copy(self.b0.at[b], self.acc.at[self.dv2.at[b]],
                              self.osem.at[b, 0]).wait()


_SC_SCRATCH = [
    pltpu.VMEM_SHARED((NACC, H), jnp.float32),
    pltpu.VMEM((2, C), jnp.int32),
    pltpu.VMEM((2, C), jnp.int32),
    pltpu.VMEM((2, C), jnp.int32),
    pltpu.VMEM((2, C), jnp.int32),
    pltpu.VMEM((2, C, H), jnp.float32),
    pltpu.VMEM((2, C, H), jnp.float32),
    pltpu.VMEM((2, C, H), jnp.float32),
    pltpu.SemaphoreType.DMA((2, 3)),
    pltpu.SemaphoreType.DMA((2, 3)),
    pltpu.SemaphoreType.DMA((2, 2)),
]


def _sc_edge_kernel(sadj_hbm, dadj_hbm, dst_hbm, w0_hbm, w1_hbm, ex_hbm,
                    zero_hbm, epre_hbm, denom_hbm,
                    acc, sa, da, dv, dv2, b0, b1, be, isem, dsem, osem):
    c = lax.axis_index("c")
    s = lax.axis_index("s")

    @pl.when(s == 0)
    def _():
        pltpu.sync_copy(zero_hbm, acc)
    plsc.subcore_barrier()

    p = _S1Pipe(c, s, sadj_hbm, dadj_hbm, dst_hbm, w0_hbm, w1_hbm, ex_hbm,
                sa, da, dv, dv2, b0, b1, be, isem, dsem, osem, acc)
    p.epre = epre_hbm
    p.run()

    plsc.subcore_barrier()

    @pl.when(s == 0)
    def _():
        pltpu.sync_copy(acc, denom_hbm.at[c])


def _sc_edge(sadj, dadj, dst, w0cat, w1cat, ex, zeros):
    return pl.kernel(
        _sc_edge_kernel,
        out_type=[jax.ShapeDtypeStruct((EP, D), jnp.float32),
                  jax.ShapeDtypeStruct((2, NACC, H), jnp.float32)],
        mesh=_MESH,
        scratch_types=_SC_SCRATCH,
    )(sadj, dadj, dst, w0cat, w1cat, ex, zeros)


def _sc_node_kernel(sadj_hbm, dadj_hbm, dst_hbm, w3_hbm, rd_hbm, ex_hbm,
                    zero_hbm, ntmp_hbm,
                    acc, sa, da, dv, dv2, b0, b1, be, isem, dsem, osem):
    c = lax.axis_index("c")
    s = lax.axis_index("s")

    @pl.when(s == 0)
    def _():
        pltpu.sync_copy(zero_hbm, acc)
    plsc.subcore_barrier()

    p = _S2Pipe(c, s, sadj_hbm, dadj_hbm, dst_hbm, w3_hbm, rd_hbm, ex_hbm,
                sa, da, dv, dv2, b0, b1, be, isem, dsem, osem, acc)
    p.run()

    plsc.subcore_barrier()

    @pl.when(s == 0)
    def _():
        pltpu.sync_copy(acc, ntmp_hbm.at[c])


def _sc_node(sadj, dadj, dst, w3cat, rdcat, ex, zeros):
    return pl.kernel(
        _sc_node_kernel,
        out_type=jax.ShapeDtypeStruct((2, NACC, H), jnp.float32),
        mesh=_MESH,
        scratch_types=_SC_SCRATCH,
    )(sadj, dadj, dst, w3cat, rdcat, ex, zeros)


def _halves_cat(x):
    return jnp.concatenate([x[:, :H], x[:, H:]], axis=0)


def kernel(n_feat, e_feat, edge_index, W0, W1, W2, W3, W4,
           gamma_e, beta_e, gamma_n, beta_n):
    src = edge_index[0]
    dst = edge_index[1]
    zp = jnp.zeros((P,), jnp.int32)
    sadj = jnp.concatenate([src, zp, src + N, zp + N])
    dadj = jnp.concatenate([dst, zp, dst + N, zp + N])
    dstp = jnp.concatenate([dst, zp + N])

    wnt = jnp.concatenate([W0, W1, W2, W3], axis=0).T
    hcat = _node_matmuls(n_feat, wnt)
    w0h, w1h, w2h, w3h = (hcat[:, :D], hcat[:, D:2 * D],
                          hcat[:, 2 * D:3 * D], hcat[:, 3 * D:])

    w4e, ex = _edge_matmul_exp(e_feat, W4.T)

    zeros = jnp.zeros((NACC, H), jnp.float32)
    e_pre, denom = _sc_edge(sadj, dadj, dstp, _halves_cat(w0h),
                            _halves_cat(w1h), ex, zeros)

    stats = _edge_stats(e_pre, w4e)
    new_e = _edge_update(stats, e_pre, w4e, e_feat,
                         gamma_e.reshape(1, D), beta_e.reshape(1, D))

    rdcat = _recip(denom[:, :N, :].reshape(2 * N, H))
    ntmp_h = _sc_node(sadj, dadj, dstp, _halves_cat(w3h), rdcat, ex, zeros)
    n_tmp = ntmp_h[:, :N, :].transpose(1, 0, 2).reshape(N, D)

    new_h = _node_update(n_tmp, w2h, n_feat,
                         gamma_n.reshape(1, D), beta_n.reshape(1, D))
    return (new_h, new_e)
